# baseline (speedup 1.0000x reference)
import functools

import jax
import jax.numpy as jnp
import numpy as np
from jax import lax
from jax.experimental import pallas as pl
from jax.experimental.pallas import tpu as pltpu
from jax.experimental.pallas import tpu_sc as plsc

_NC = 2
_NS = 16
_L = 16

_B = 16384
_D = 16
_N = 1000000
_H = _N // 2
_BPT = _B // _NS
_DPC = _D // _NC
_STEPS = _DPC * 2
_SENT = -1


def _sc_body(iu_hbm, iv_hbm, tab3_hbm, sig_hbm, d2p_hbm, ssp_hbm,
             iu_v, iv_v, imu0, imu1, mfu, imv0, imv1, mfv,
             lu0, lu1, lv0, lv1, d2a, ssa, sig_v16,
             ring_sh, dsem, gsem):
    cid = lax.axis_index("c")
    sid = lax.axis_index("s")
    pltpu.sync_copy(iu_hbm.at[sid], iu_v)
    pltpu.sync_copy(iv_hbm.at[sid], iv_v)
    pltpu.sync_copy(sig_hbm, sig_v16)

    half = jnp.full((_L,), _H, dtype=jnp.int32)
    sent = jnp.full((_L,), _SENT, dtype=jnp.int32)

    def prep(j, carry):
        o = pl.multiple_of(j * _L, _L)
        nu = iu_v[pl.ds(o, _L)]
        tu = nu >= half
        imu0[pl.ds(o, _L)] = jnp.where(tu, sent, nu)
        imu1[pl.ds(o, _L)] = jnp.where(tu, nu - half, sent)
        mfu[pl.ds(o, _L)] = jnp.where(tu, 1.0, 0.0)
        nv = iv_v[pl.ds(o, _L)]
        tv = nv >= half
        imv0[pl.ds(o, _L)] = jnp.where(tv, sent, nv)
        imv1[pl.ds(o, _L)] = jnp.where(tv, nv - half, sent)
        mfv[pl.ds(o, _L)] = jnp.where(tv, 1.0, 0.0)
        return carry

    lax.fori_loop(0, _BPT // _L, prep, 0)

    @pl.when(sid == 0)
    def _stage0():
        pltpu.sync_copy(tab3_hbm.at[cid, 0, pl.ds(0, _H)], ring_sh.at[0])

    plsc.subcore_barrier()

    lu = [lu0, lu1]
    lv = [lv0, lv1]
    imus = [imu0, imu1]
    imvs = [imv0, imv1]

    for s in range(_STEPS):
        k, h = divmod(s, 2)
        if s + 1 < _STEPS:
            kn, hn = divmod(s + 1, 2)

            @pl.when(sid == 0)
            def _start_next(kn=kn, hn=hn, s=s):
                pltpu.async_copy(tab3_hbm.at[cid, kn, pl.ds(hn * _H, _H)],
                                 ring_sh.at[(s + 1) % 2], dsem)

        buf = ring_sh.at[s % 2]
        cps = [
            pltpu.async_copy(
                buf.at[plsc.Indices(imus[h], ignored_value=_SENT)], lu[h], gsem),
            pltpu.async_copy(
                buf.at[plsc.Indices(imvs[h], ignored_value=_SENT)], lv[h], gsem),
        ]
        for cp in cps:
            cp.wait()

        if h == 1:
            ones = jnp.ones((_L,), jnp.float32)
            dcol = jnp.full((_L,), k, dtype=jnp.int32) + cid * _DPC
            sigk = ones / plsc.load_gather(sig_v16, [dcol])

            def acc(g, carry, k=k, sigk=sigk):
                o = pl.multiple_of(g * _L, _L)
                u0 = lu0[pl.ds(o, _L)]
                u = u0 + (lu1[pl.ds(o, _L)] - u0) * mfu[pl.ds(o, _L)]
                v0 = lv0[pl.ds(o, _L)]
                v = v0 + (lv1[pl.ds(o, _L)] - v0) * mfv[pl.ds(o, _L)]
                diff = u - v
                dterm = diff * diff
                sterm = (u * u + v * v) * sigk
                if k == 0:
                    d2a[pl.ds(o, _L)] = dterm
                    ssa[pl.ds(o, _L)] = sterm
                else:
                    d2a[pl.ds(o, _L)] = d2a[pl.ds(o, _L)] + dterm
                    ssa[pl.ds(o, _L)] = ssa[pl.ds(o, _L)] + sterm
                return carry

            lax.fori_loop(0, _BPT // _L, acc, 0)

        if s + 1 < _STEPS:
            kn, hn = divmod(s + 1, 2)

            @pl.when(sid == 0)
            def _drain_next(kn=kn, hn=hn, s=s):
                pltpu.make_async_copy(
                    tab3_hbm.at[cid, kn, pl.ds(hn * _H, _H)],
                    ring_sh.at[(s + 1) % 2], dsem).wait()

        plsc.subcore_barrier()

    base = sid * _BPT
    pltpu.sync_copy(d2a, d2p_hbm.at[cid, pl.ds(base, _BPT)])
    pltpu.sync_copy(ssa, ssp_hbm.at[cid, pl.ds(base, _BPT)])


@functools.cache
def _make_sc_call():
    @functools.partial(
        pl.kernel,
        mesh=plsc.VectorSubcoreMesh(core_axis_name="c", subcore_axis_name="s"),
        compiler_params=pltpu.CompilerParams(
            needs_layout_passes=False, use_tc_tiling_on_sc=False),
        out_type=[
            jax.ShapeDtypeStruct((_NC, _B), jnp.float32),
            jax.ShapeDtypeStruct((_NC, _B), jnp.float32),
        ],
        scratch_types=[
            pltpu.VMEM((_BPT,), jnp.int32),
            pltpu.VMEM((_BPT,), jnp.int32),
            pltpu.VMEM((_BPT,), jnp.int32),
            pltpu.VMEM((_BPT,), jnp.int32),
            pltpu.VMEM((_BPT,), jnp.float32),
            pltpu.VMEM((_BPT,), jnp.int32),
            pltpu.VMEM((_BPT,), jnp.int32),
            pltpu.VMEM((_BPT,), jnp.float32),
            pltpu.VMEM((_BPT,), jnp.float32),
            pltpu.VMEM((_BPT,), jnp.float32),
            pltpu.VMEM((_BPT,), jnp.float32),
            pltpu.VMEM((_BPT,), jnp.float32),
            pltpu.VMEM((_BPT,), jnp.float32),
            pltpu.VMEM((_BPT,), jnp.float32),
            pltpu.VMEM((_D,), jnp.float32),
            pltpu.VMEM_SHARED((2, _H), jnp.float32),
            pltpu.SemaphoreType.DMA,
            pltpu.SemaphoreType.DMA,
        ],
    )
    def _sc_call(iu_hbm, iv_hbm, tab3_hbm, sig_hbm, d2p_hbm, ssp_hbm, *scratch):
        _sc_body(iu_hbm, iv_hbm, tab3_hbm, sig_hbm, d2p_hbm, ssp_hbm, *scratch)

    return _sc_call


def _tc_body(bg_ref, sig_ref, d2_ref, s_ref, lab_ref, out_ref):
    beta = bg_ref[0]
    gamma = bg_ref[1]
    const2 = _D * jnp.log(jnp.float32(2.0 * np.pi)) + jnp.sum(jnp.log(sig_ref[...]))
    pd = d2_ref[...]
    ps = s_ref[...]
    d2 = pd[0:1, :] + pd[1:2, :]
    ss = ps[0:1, :] + ps[1:2, :]
    dist = jnp.sqrt(d2)
    x = beta * dist - gamma
    sp = jnp.maximum(x, 0.0) + jnp.log1p(jnp.exp(-jnp.abs(x)))
    sn = sp - x
    latent = (const2 + 0.5 * ss) * jnp.float32(1.0 / (_N - 1))
    out_ref[...] = jnp.where(lab_ref[...] == 1, sp, sn) + latent


def _tc_call(bg, sig, d2p, ssp, lab):
    return pl.pallas_call(
        _tc_body,
        out_shape=jax.ShapeDtypeStruct((1, _B), jnp.float32),
        in_specs=[
            pl.BlockSpec(memory_space=pltpu.SMEM),
            pl.BlockSpec(memory_space=pltpu.VMEM),
            pl.BlockSpec(memory_space=pltpu.VMEM),
            pl.BlockSpec(memory_space=pltpu.VMEM),
            pl.BlockSpec(memory_space=pltpu.VMEM),
        ],
    )(bg, sig, d2p, ssp, lab)


def kernel(pairs, labels, table, sigma, beta, gamma):
    iu = pairs[:, 0].reshape(_NS, _BPT)
    iv = pairs[:, 1].reshape(_NS, _BPT)
    tab3 = table.T.reshape(_NC, _DPC, _N)
    d2p, ssp = _make_sc_call()(iu, iv, tab3, sigma)
    bg = jnp.stack([beta, gamma]).astype(jnp.float32)
    loss = _tc_call(bg, sigma.reshape(1, _D), d2p, ssp, labels.reshape(1, _B))
    return loss.reshape(_B)


# speedup vs baseline: 2.8483x; 2.8483x over previous
import functools

import jax
import jax.numpy as jnp
import numpy as np
from jax import lax
from jax.experimental import pallas as pl
from jax.experimental.pallas import tpu as pltpu
from jax.experimental.pallas import tpu_sc as plsc

_NC = 2
_NS = 16
_NW = _NC * _NS
_L = 16

_B = 16384
_D = 16
_BPW = _B // _NW
_CH = _BPW // 128
_HB = _BPW // 2
_GROUPS = _HB // _L


def _sc_body(iu_hbm, iu2_hbm, iv_hbm, iv2_hbm, table_hbm, sig_hbm, d2_hbm, s_hbm,
             idxu_v, idxu2_v, idxv_v, idxv2_v, us_v, vs_v, sig_v, sigb_v,
             d2_v, s_v, sem):
    wid = lax.axis_index("s") * _NC + lax.axis_index("c")
    base = wid * _BPW
    pltpu.sync_copy(iu_hbm.at[wid], idxu_v)
    pltpu.sync_copy(iu2_hbm.at[wid], idxu2_v)
    pltpu.sync_copy(iv_hbm.at[wid], idxv_v)
    pltpu.sync_copy(iv2_hbm.at[wid], idxv2_v)
    pltpu.sync_copy(sig_hbm, sig_v)
    ones = jnp.ones((_L,), jnp.float32)
    sig_vec = sig_v[...]
    for d in range(_D):
        sigb_v[pl.ds(d * _L, _L)] = ones / (sig_vec[d] * ones)

    iota = lax.iota(jnp.int32, _L)
    seven = jnp.full((_L,), 7, dtype=jnp.int32)

    for half in range(2):
        copies = []
        for c in range(_HB // 128):
            cc = half * (_HB // 128) + c
            copies.append(pltpu.async_copy(
                table_hbm.at[idxu2_v.at[cc]], us_v.at[pl.ds(c * 128, 128)],
                sem))
            copies.append(pltpu.async_copy(
                table_hbm.at[idxv2_v.at[cc]], vs_v.at[pl.ds(c * 128, 128)],
                sem))
        for cp in copies:
            cp.wait()

        def group(g, carry, half=half):
            rows = g * _L + iota
            gc = half * (_HB // 128) + g // 8
            go = (g % 8) * _L
            nu = idxu_v[gc, pl.ds(go, _L)]
            nv = idxv_v[gc, pl.ds(go, _L)]
            ubase = (nu & seven) * _L
            vbase = (nv & seven) * _L
            d2 = jnp.zeros((_L,), jnp.float32)
            ss = jnp.zeros((_L,), jnp.float32)
            for d in range(_D):
                tu = plsc.load_gather(us_v, [rows, ubase + d])
                tv = plsc.load_gather(vs_v, [rows, vbase + d])
                diff = tu - tv
                d2 = d2 + diff * diff
                ss = ss + (tu * tu + tv * tv) * sigb_v[pl.ds(d * _L, _L)]
            off = pl.multiple_of(half * _HB + g * _L, _L)
            d2_v[pl.ds(off, _L)] = d2
            s_v[pl.ds(off, _L)] = ss
            return carry

        lax.fori_loop(0, _GROUPS, group, 0)

    pltpu.sync_copy(d2_v, d2_hbm.at[pl.ds(base, _BPW)])
    pltpu.sync_copy(s_v, s_hbm.at[pl.ds(base, _BPW)])


@functools.cache
def _make_sc_call():
    @functools.partial(
        pl.kernel,
        mesh=plsc.VectorSubcoreMesh(core_axis_name="c", subcore_axis_name="s"),
        compiler_params=pltpu.CompilerParams(
            needs_layout_passes=False, use_tc_tiling_on_sc=False),
        out_type=[
            jax.ShapeDtypeStruct((_B,), jnp.float32),
            jax.ShapeDtypeStruct((_B,), jnp.float32),
        ],
        scratch_types=[
            pltpu.VMEM((_CH, 128), jnp.int32),
            pltpu.VMEM((_CH, 128), jnp.int32),
            pltpu.VMEM((_CH, 128), jnp.int32),
            pltpu.VMEM((_CH, 128), jnp.int32),
            pltpu.VMEM((_HB, 128), jnp.float32),
            pltpu.VMEM((_HB, 128), jnp.float32),
            pltpu.VMEM((_D,), jnp.float32),
            pltpu.VMEM((_D * _L,), jnp.float32),
            pltpu.VMEM((_BPW,), jnp.float32),
            pltpu.VMEM((_BPW,), jnp.float32),
            pltpu.SemaphoreType.DMA,
        ],
    )
    def _sc_call(iu_hbm, iu2_hbm, iv_hbm, iv2_hbm, table_hbm, sig_hbm,
                 d2_hbm, s_hbm, *scratch):
        _sc_body(iu_hbm, iu2_hbm, iv_hbm, iv2_hbm, table_hbm, sig_hbm,
                 d2_hbm, s_hbm, *scratch)

    return _sc_call


def _tc_body(bg_ref, sig_ref, d2_ref, s_ref, lab_ref, out_ref):
    beta = bg_ref[0]
    gamma = bg_ref[1]
    const2 = _D * jnp.log(jnp.float32(2.0 * np.pi)) + jnp.sum(jnp.log(sig_ref[...]))
    dist = jnp.sqrt(d2_ref[...])
    x = beta * dist - gamma
    sp = jnp.maximum(x, 0.0) + jnp.log1p(jnp.exp(-jnp.abs(x)))
    sn = sp - x
    latent = (const2 + 0.5 * s_ref[...]) * jnp.float32(1.0 / (1000000 - 1))
    out_ref[...] = jnp.where(lab_ref[...] == 1, sp, sn) + latent


def _tc_call(bg, sig, d2, ss, lab):
    return pl.pallas_call(
        _tc_body,
        out_shape=jax.ShapeDtypeStruct((128, 128), jnp.float32),
        in_specs=[
            pl.BlockSpec(memory_space=pltpu.SMEM),
            pl.BlockSpec(memory_space=pltpu.VMEM),
            pl.BlockSpec(memory_space=pltpu.VMEM),
            pl.BlockSpec(memory_space=pltpu.VMEM),
            pl.BlockSpec(memory_space=pltpu.VMEM),
        ],
    )(bg, sig, d2, ss, lab)


def kernel(pairs, labels, table, sigma, beta, gamma):
    iu = pairs[:, 0].reshape(_NW, _CH, 128)
    iv = pairs[:, 1].reshape(_NW, _CH, 128)
    iu2 = (pairs[:, 0] >> 3).reshape(_NW, _CH, 128)
    iv2 = (pairs[:, 1] >> 3).reshape(_NW, _CH, 128)
    tab2 = table.reshape(125000, 128)
    d2, ss = _make_sc_call()(iu, iu2, iv, iv2, tab2, sigma)
    bg = jnp.stack([beta, gamma]).astype(jnp.float32)
    loss = _tc_call(bg, sigma.reshape(1, _D), d2.reshape(128, 128),
                    ss.reshape(128, 128), labels.reshape(128, 128))
    return loss.reshape(_B)


# speedup vs baseline: 2.8839x; 1.0125x over previous
import functools

import jax
import jax.numpy as jnp
import numpy as np
from jax import lax
from jax.experimental import pallas as pl
from jax.experimental.pallas import tpu as pltpu
from jax.experimental.pallas import tpu_sc as plsc

_NC = 2
_NS = 16
_NW = _NC * _NS
_L = 16

_B = 16384
_D = 16
_BPW = _B // _NW
_CH = _BPW // 128
_GROUPS = _BPW // _L


def _sc_body(iu_hbm, iv_hbm, table_hbm, sig_hbm, d2_hbm, s_hbm,
             idxu_v, idxv_v, us_v, vs_v, sig_v, sigb_v, d2_v, s_v, sem):
    wid = lax.axis_index("s") * _NC + lax.axis_index("c")
    base = wid * _BPW
    pltpu.sync_copy(iu_hbm.at[wid], idxu_v)
    pltpu.sync_copy(iv_hbm.at[wid], idxv_v)
    pltpu.sync_copy(sig_hbm, sig_v)
    copies = []
    for c in range(_CH):
        copies.append(pltpu.async_copy(
            table_hbm.at[idxu_v.at[c]], us_v.at[pl.ds(c * 128, 128)], sem))
        copies.append(pltpu.async_copy(
            table_hbm.at[idxv_v.at[c]], vs_v.at[pl.ds(c * 128, 128)], sem))
    ones = jnp.ones((_L,), jnp.float32)
    sig_vec = sig_v[...]
    for d in range(_D):
        sigb_v[pl.ds(d * _L, _L)] = ones / (sig_vec[d] * ones)
    for cp in copies:
        cp.wait()

    iota = lax.iota(jnp.int32, _L)

    def group(g, carry):
        rows = g * _L + iota
        d2 = jnp.zeros((_L,), jnp.float32)
        ss = jnp.zeros((_L,), jnp.float32)
        for d in range(_D):
            cols = jnp.full((_L,), d, dtype=jnp.int32)
            tu = plsc.load_gather(us_v, [rows, cols])
            tv = plsc.load_gather(vs_v, [rows, cols])
            diff = tu - tv
            d2 = d2 + diff * diff
            ss = ss + (tu * tu + tv * tv) * sigb_v[pl.ds(d * _L, _L)]
        off = pl.multiple_of(g * _L, _L)
        d2_v[pl.ds(off, _L)] = d2
        s_v[pl.ds(off, _L)] = ss
        return carry

    lax.fori_loop(0, _GROUPS, group, 0)
    pltpu.sync_copy(d2_v, d2_hbm.at[pl.ds(base, _BPW)])
    pltpu.sync_copy(s_v, s_hbm.at[pl.ds(base, _BPW)])


@functools.cache
def _make_sc_call():
    @functools.partial(
        pl.kernel,
        mesh=plsc.VectorSubcoreMesh(core_axis_name="c", subcore_axis_name="s"),
        compiler_params=pltpu.CompilerParams(
            needs_layout_passes=False, use_tc_tiling_on_sc=False),
        out_type=[
            jax.ShapeDtypeStruct((_B,), jnp.float32),
            jax.ShapeDtypeStruct((_B,), jnp.float32),
        ],
        scratch_types=[
            pltpu.VMEM((_CH, 128), jnp.int32),
            pltpu.VMEM((_CH, 128), jnp.int32),
            pltpu.VMEM((_BPW, _D), jnp.float32),
            pltpu.VMEM((_BPW, _D), jnp.float32),
            pltpu.VMEM((_D,), jnp.float32),
            pltpu.VMEM((_D * _L,), jnp.float32),
            pltpu.VMEM((_BPW,), jnp.float32),
            pltpu.VMEM((_BPW,), jnp.float32),
            pltpu.SemaphoreType.DMA,
        ],
    )
    def _sc_call(iu_hbm, iv_hbm, table_hbm, sig_hbm, d2_hbm, s_hbm, *scratch):
        _sc_body(iu_hbm, iv_hbm, table_hbm, sig_hbm, d2_hbm, s_hbm, *scratch)

    return _sc_call


def _tc_body(bg_ref, sig_ref, d2_ref, s_ref, lab_ref, out_ref):
    beta = bg_ref[0]
    gamma = bg_ref[1]
    const2 = _D * jnp.log(jnp.float32(2.0 * np.pi)) + jnp.sum(jnp.log(sig_ref[...]))
    dist = jnp.sqrt(d2_ref[...])
    x = beta * dist - gamma
    sp = jnp.maximum(x, 0.0) + jnp.log1p(jnp.exp(-jnp.abs(x)))
    sn = sp - x
    latent = (const2 + 0.5 * s_ref[...]) * jnp.float32(1.0 / (1000000 - 1))
    out_ref[...] = jnp.where(lab_ref[...] == 1, sp, sn) + latent


def _tc_call(bg, sig, d2, ss, lab):
    return pl.pallas_call(
        _tc_body,
        out_shape=jax.ShapeDtypeStruct((128, 128), jnp.float32),
        in_specs=[
            pl.BlockSpec(memory_space=pltpu.SMEM),
            pl.BlockSpec(memory_space=pltpu.VMEM),
            pl.BlockSpec(memory_space=pltpu.VMEM),
            pl.BlockSpec(memory_space=pltpu.VMEM),
            pl.BlockSpec(memory_space=pltpu.VMEM),
        ],
    )(bg, sig, d2, ss, lab)


def kernel(pairs, labels, table, sigma, beta, gamma):
    iu = pairs[:, 0].reshape(_NW, _CH, 128)
    iv = pairs[:, 1].reshape(_NW, _CH, 128)
    d2, ss = _make_sc_call()(iu, iv, table, sigma)
    bg = jnp.stack([beta, gamma]).astype(jnp.float32)
    loss = _tc_call(bg, sigma.reshape(1, _D), d2.reshape(128, 128),
                    ss.reshape(128, 128), labels.reshape(128, 128))
    return loss.reshape(_B)
